# baseline (speedup 1.0000x reference)
import functools

import jax
import jax.numpy as jnp
from jax import lax
from jax.experimental import pallas as pl
from jax.experimental.pallas import tpu as pltpu
from jax.experimental.pallas import tpu_sc as plsc

N = 10000
D = 128
E = 320000
NC = 2
NS = 16
NPAD = 10112
ROWS_PER_TILE_OUT = NPAD // NS
EPAD = 327680
EROWS = EPAD // 128
ROWS_PER_TILE = EROWS // (NC * NS)
OUTER = ROWS_PER_TILE // 8
R0 = 136
R1 = 24
MAXOUTER = max(R0, R1) // 8

_mesh = plsc.VectorSubcoreMesh(core_axis_name="c", subcore_axis_name="s")


def _fill_buf_2d(buf, nrows, ncols, value):
    @pl.loop(0, nrows)
    def _(i):
        @pl.loop(0, ncols, step=16)
        def _(j):
            buf.at[pl.ds(i, 1), pl.ds(j, 16)][...] = jnp.full(
                (1, 16), value, jnp.float32)


@functools.partial(
    pl.kernel,
    out_type=jax.ShapeDtypeStruct((NC, NPAD, 128), jnp.float32),
    mesh=_mesh,
    scratch_types=[
        pltpu.VMEM((8, 128), jnp.int32),
        pltpu.VMEM((128, 128), jnp.float32),
        pltpu.VMEM((128, 128), jnp.float32),
        pltpu.SemaphoreType.DMA,
        pltpu.VMEM_SHARED((NPAD, 128), jnp.float32),
    ],
)
def _sc_degree(dstm_hbm, out_hbm, dstv, ones, zeros, hsem, acc):
    c = lax.axis_index("c")
    s = lax.axis_index("s")
    wid = c * NS + s
    _fill_buf_2d(ones, 128, 128, 1.0)
    _fill_buf_2d(zeros, 128, 128, 0.0)
    base = s * ROWS_PER_TILE_OUT
    for off, cnt in ((0, 128), (128, 128), (256, 128), (384, 128), (512, 120)):
        pltpu.sync_copy(zeros.at[pl.ds(0, cnt)], acc.at[pl.ds(base + off, cnt)])
    plsc.subcore_barrier()

    base_row = wid * ROWS_PER_TILE

    @pl.loop(0, OUTER)
    def _(it):
        r0 = base_row + it * 8
        pltpu.sync_copy(dstm_hbm.at[pl.ds(r0, 8)], dstv)
        descs = [pltpu.async_copy(ones, acc.at[dstv.at[j]], hsem, add=True)
                 for j in range(8)]
        for d in descs:
            d.wait()

    plsc.subcore_barrier()
    pltpu.sync_copy(acc.at[pl.ds(base, ROWS_PER_TILE_OUT)],
                    out_hbm.at[c].at[pl.ds(base, ROWS_PER_TILE_OUT)])


def _make_msgpass(fav):
  @functools.partial(
      pl.kernel,
      out_type=jax.ShapeDtypeStruct((NC, NPAD, D), jnp.float32),
      mesh=_mesh,
      scratch_types=[
          pltpu.VMEM((8, 128), jnp.int32),
          pltpu.VMEM((8, 128), jnp.int32),
          pltpu.VMEM((128, D), jnp.float32),
          pltpu.VMEM((128, D), jnp.float32),
          pltpu.SemaphoreType.DMA((2,)),
          pltpu.SemaphoreType.DMA((2,)),
          pltpu.VMEM_SHARED((NPAD, D), jnp.float32),
      ],
  )
  def _sc_msgpass(q_hbm, srcm_hbm, dstm_hbm, out_hbm, srcv, dstv, b0, b1,
                  gsem, ssem, acc):
      bufs = [b0, b1]
      c = lax.axis_index("c")
      s = lax.axis_index("s")
      wid = c * NS + s
      _fill_buf_2d(b0, 128, D, 0.0)
      base = s * ROWS_PER_TILE_OUT
      for off, cnt in ((0, 128), (128, 128), (256, 128), (384, 128), (512, 120)):
          pltpu.sync_copy(b0.at[pl.ds(0, cnt)], acc.at[pl.ds(base + off, cnt)])
      plsc.subcore_barrier()

      base_row = jnp.where(c == 0, s * R0, 16 * R0 + s * R1)
      n_outer = jnp.where(c == 0, R0 // 8, R1 // 8)

      @pl.loop(0, MAXOUTER)
      def _(it):
          @pl.when(it < n_outer)
          def _():
              r0 = base_row + it * 8
              pltpu.sync_copy(srcm_hbm.at[pl.ds(r0, 8)], srcv)
              pltpu.sync_copy(dstm_hbm.at[pl.ds(r0, 8)], dstv)
              g = {0: pltpu.async_copy(q_hbm.at[srcv.at[0]], bufs[0],
                                       gsem.at[0])}
              s_ = {}
              for j in range(8):
                  b = j & 1
                  if j >= 1:
                      s_[j - 1].wait()
                  if j < 7:
                      g[j + 1] = pltpu.async_copy(q_hbm.at[srcv.at[j + 1]],
                                                  bufs[1 - b], gsem.at[1 - b])
                  g[j].wait()
                  s_[j] = pltpu.async_copy(bufs[b], acc.at[dstv.at[j]],
                                           ssem.at[b], add=True)
              s_[7].wait()

      plsc.subcore_barrier()
      pltpu.sync_copy(acc.at[pl.ds(base, ROWS_PER_TILE_OUT)],
                      out_hbm.at[c].at[pl.ds(base, ROWS_PER_TILE_OUT)])


  return _sc_msgpass


_sc_msgpass_a = _make_msgpass(0)
_sc_msgpass_b = _make_msgpass(1)


def _dinv_from_degp(degp):
    deg = degp[0, :, 0:1] + degp[1, :, 0:1] + 1.0
    return lax.rsqrt(deg)


def _mm_body(x_ref, w_ref, o_ref):
    o_ref[...] = jnp.dot(x_ref[...], w_ref[...],
                         preferred_element_type=jnp.float32)


def _q_body(degp_ref, p_ref, q_ref):
    dinv = _dinv_from_degp(degp_ref[...])
    rid = lax.broadcasted_iota(jnp.int32, (NPAD, 1), 0)
    q_ref[...] = jnp.where(rid < N, dinv * p_ref[...], 0.0)


def _comb1_body(degp_ref, s_ref, p_ref, b1_ref, w2_ref, r_ref, q2_ref):
    dinv = _dinv_from_degp(degp_ref[...])
    m = (dinv * (s_ref[0] + s_ref[1]) + (dinv * dinv) * p_ref[...]
         + b1_ref[...])
    h = jnp.maximum(m, 0.0)
    r = jnp.dot(h, w2_ref[...], preferred_element_type=jnp.float32)
    r_ref[...] = r
    rid = lax.broadcasted_iota(jnp.int32, (NPAD, 1), 0)
    q2_ref[...] = jnp.where(rid < N, dinv * r, 0.0)


def _comb2_body(degp_ref, s_ref, r_ref, b2_ref, o_ref):
    dinv = _dinv_from_degp(degp_ref[...])
    o_ref[...] = (dinv * (s_ref[0] + s_ref[1])
                  + (dinv * dinv) * r_ref[...] + b2_ref[...])


_f32 = jnp.float32
_mm = pl.pallas_call(_mm_body, out_shape=jax.ShapeDtypeStruct((NPAD, D), _f32))
_qk = pl.pallas_call(_q_body, out_shape=jax.ShapeDtypeStruct((NPAD, D), _f32))
_comb1 = pl.pallas_call(
    _comb1_body,
    out_shape=(jax.ShapeDtypeStruct((NPAD, D), _f32),
               jax.ShapeDtypeStruct((NPAD, D), _f32)))
_comb2 = pl.pallas_call(
    _comb2_body, out_shape=jax.ShapeDtypeStruct((NPAD, D), _f32))


def kernel(x, edge_index, W1, b1, W2, b2):
    src = edge_index[0].astype(jnp.int32)
    dst = edge_index[1].astype(jnp.int32)
    pad = jnp.full((EPAD - E,), N, jnp.int32)
    srcm = jnp.concatenate([src, pad]).reshape(EROWS, 128)
    dstm = jnp.concatenate([dst, pad]).reshape(EROWS, 128)
    xpad = jnp.concatenate([x, jnp.zeros((NPAD - N, D), _f32)], axis=0)
    b1r = b1.reshape(1, D)
    b2r = b2.reshape(1, D)

    degp = _sc_degree(dstm)
    p = _mm(xpad, W1)
    q = _qk(degp, p)
    s1 = _sc_msgpass_a(q, srcm, dstm)
    r, q2 = _comb1(degp, s1, p, b1r, W2)
    s2 = _sc_msgpass_b(q2, srcm, dstm)
    out = _comb2(degp, s2, r, b2r)
    return out[:N]


# speedup vs baseline: 1.1524x; 1.1524x over previous
import functools

import jax
import jax.numpy as jnp
from jax import lax
from jax.experimental import pallas as pl
from jax.experimental.pallas import tpu as pltpu
from jax.experimental.pallas import tpu_sc as plsc

N = 10000
D = 128
E = 320000
NC = 2
NS = 16
NPAD = 10112
ROWS_PER_TILE_OUT = NPAD // NS
EPAD = 327680
EROWS = EPAD // 128
ROWS_PER_TILE = EROWS // (NC * NS)
OUTER = ROWS_PER_TILE // 8
R0 = 152
R1 = 8
MAXOUTER = max(R0, R1) // 8

_mesh = plsc.VectorSubcoreMesh(core_axis_name="c", subcore_axis_name="s")


def _fill_buf_2d(buf, nrows, ncols, value):
    @pl.loop(0, nrows)
    def _(i):
        @pl.loop(0, ncols, step=16)
        def _(j):
            buf.at[pl.ds(i, 1), pl.ds(j, 16)][...] = jnp.full(
                (1, 16), value, jnp.float32)


@functools.partial(
    pl.kernel,
    out_type=jax.ShapeDtypeStruct((NC, NPAD, 128), jnp.float32),
    mesh=_mesh,
    scratch_types=[
        pltpu.VMEM((8, 128), jnp.int32),
        pltpu.VMEM((128, 128), jnp.float32),
        pltpu.VMEM((128, 128), jnp.float32),
        pltpu.SemaphoreType.DMA,
        pltpu.VMEM_SHARED((NPAD, 128), jnp.float32),
    ],
)
def _sc_degree(dstm_hbm, out_hbm, dstv, ones, zeros, hsem, acc):
    c = lax.axis_index("c")
    s = lax.axis_index("s")
    wid = c * NS + s
    _fill_buf_2d(ones, 128, 128, 1.0)
    _fill_buf_2d(zeros, 128, 128, 0.0)
    base = s * ROWS_PER_TILE_OUT
    for off, cnt in ((0, 128), (128, 128), (256, 128), (384, 128), (512, 120)):
        pltpu.sync_copy(zeros.at[pl.ds(0, cnt)], acc.at[pl.ds(base + off, cnt)])
    plsc.subcore_barrier()

    base_row = wid * ROWS_PER_TILE

    @pl.loop(0, OUTER)
    def _(it):
        r0 = base_row + it * 8
        pltpu.sync_copy(dstm_hbm.at[pl.ds(r0, 8)], dstv)
        descs = [pltpu.async_copy(ones, acc.at[dstv.at[j]], hsem, add=True)
                 for j in range(8)]
        for d in descs:
            d.wait()

    plsc.subcore_barrier()
    pltpu.sync_copy(acc.at[pl.ds(base, ROWS_PER_TILE_OUT)],
                    out_hbm.at[c].at[pl.ds(base, ROWS_PER_TILE_OUT)])


def _make_msgpass(fav):
  @functools.partial(
      pl.kernel,
      out_type=jax.ShapeDtypeStruct((NC, NPAD, D), jnp.float32),
      mesh=_mesh,
      scratch_types=[
          pltpu.VMEM((8, 128), jnp.int32),
          pltpu.VMEM((8, 128), jnp.int32),
          pltpu.VMEM((128, D), jnp.float32),
          pltpu.VMEM((128, D), jnp.float32),
          pltpu.SemaphoreType.DMA((2,)),
          pltpu.SemaphoreType.DMA((2,)),
          pltpu.VMEM_SHARED((NPAD, D), jnp.float32),
      ],
  )
  def _sc_msgpass(q_hbm, srcm_hbm, dstm_hbm, out_hbm, srcv, dstv, b0, b1,
                  gsem, ssem, acc):
      bufs = [b0, b1]
      c = lax.axis_index("c")
      s = lax.axis_index("s")
      wid = c * NS + s
      _fill_buf_2d(b0, 128, D, 0.0)
      base = s * ROWS_PER_TILE_OUT
      for off, cnt in ((0, 128), (128, 128), (256, 128), (384, 128), (512, 120)):
          pltpu.sync_copy(b0.at[pl.ds(0, cnt)], acc.at[pl.ds(base + off, cnt)])
      plsc.subcore_barrier()

      base_row = jnp.where(c == 0, s * R0, 16 * R0 + s * R1)
      n_outer = jnp.where(c == 0, R0 // 8, R1 // 8)

      @pl.loop(0, MAXOUTER)
      def _(it):
          @pl.when(it < n_outer)
          def _():
              r0 = base_row + it * 8
              pltpu.sync_copy(srcm_hbm.at[pl.ds(r0, 8)], srcv)
              pltpu.sync_copy(dstm_hbm.at[pl.ds(r0, 8)], dstv)
              g = {0: pltpu.async_copy(q_hbm.at[srcv.at[0]], bufs[0],
                                       gsem.at[0])}
              s_ = {}
              for j in range(8):
                  b = j & 1
                  if j >= 1:
                      s_[j - 1].wait()
                  if j < 7:
                      g[j + 1] = pltpu.async_copy(q_hbm.at[srcv.at[j + 1]],
                                                  bufs[1 - b], gsem.at[1 - b])
                  g[j].wait()
                  s_[j] = pltpu.async_copy(bufs[b], acc.at[dstv.at[j]],
                                           ssem.at[b], add=True)
              s_[7].wait()

      plsc.subcore_barrier()
      pltpu.sync_copy(acc.at[pl.ds(base, ROWS_PER_TILE_OUT)],
                      out_hbm.at[c].at[pl.ds(base, ROWS_PER_TILE_OUT)])


  return _sc_msgpass


_sc_msgpass_a = _make_msgpass(0)
_sc_msgpass_b = _make_msgpass(1)


def _dinv_from_degp(degp):
    deg = degp[0, :, 0:1] + degp[1, :, 0:1] + 1.0
    return lax.rsqrt(deg)


def _mm_body(x_ref, w_ref, o_ref):
    o_ref[...] = jnp.dot(x_ref[...], w_ref[...],
                         preferred_element_type=jnp.float32)


def _q_body(degp_ref, p_ref, q_ref):
    dinv = _dinv_from_degp(degp_ref[...])
    rid = lax.broadcasted_iota(jnp.int32, (NPAD, 1), 0)
    q_ref[...] = jnp.where(rid < N, dinv * p_ref[...], 0.0)


def _comb1_body(degp_ref, s_ref, p_ref, b1_ref, w2_ref, r_ref, q2_ref):
    dinv = _dinv_from_degp(degp_ref[...])
    m = (dinv * (s_ref[0] + s_ref[1]) + (dinv * dinv) * p_ref[...]
         + b1_ref[...])
    h = jnp.maximum(m, 0.0)
    r = jnp.dot(h, w2_ref[...], preferred_element_type=jnp.float32)
    r_ref[...] = r
    rid = lax.broadcasted_iota(jnp.int32, (NPAD, 1), 0)
    q2_ref[...] = jnp.where(rid < N, dinv * r, 0.0)


def _comb2_body(degp_ref, s_ref, r_ref, b2_ref, o_ref):
    dinv = _dinv_from_degp(degp_ref[...])
    val = (dinv * (s_ref[0] + s_ref[1])
           + (dinv * dinv) * r_ref[...] + b2_ref[...])
    o_ref[...] = val[:N]


_f32 = jnp.float32
_mm = pl.pallas_call(_mm_body, out_shape=jax.ShapeDtypeStruct((NPAD, D), _f32))
_qk = pl.pallas_call(_q_body, out_shape=jax.ShapeDtypeStruct((NPAD, D), _f32))
_comb1 = pl.pallas_call(
    _comb1_body,
    out_shape=(jax.ShapeDtypeStruct((NPAD, D), _f32),
               jax.ShapeDtypeStruct((NPAD, D), _f32)))
_comb2 = pl.pallas_call(
    _comb2_body, out_shape=jax.ShapeDtypeStruct((N, D), _f32))


def kernel(x, edge_index, W1, b1, W2, b2):
    src = edge_index[0].astype(jnp.int32)
    dst = edge_index[1].astype(jnp.int32)
    pad = jnp.full((EPAD - E,), N, jnp.int32)
    srcm = jnp.concatenate([src, pad]).reshape(EROWS, 128)
    dstm = jnp.concatenate([dst, pad]).reshape(EROWS, 128)
    xpad = jnp.concatenate([x, jnp.zeros((NPAD - N, D), _f32)], axis=0)
    b1r = b1.reshape(1, D)
    b2r = b2.reshape(1, D)

    degp = _sc_degree(dstm)
    p = _mm(xpad, W1)
    q = _qk(degp, p)
    s1 = _sc_msgpass_a(q, srcm, dstm)
    r, q2 = _comb1(degp, s1, p, b1r, W2)
    s2 = _sc_msgpass_b(q2, srcm, dstm)
    return _comb2(degp, s2, r, b2r)
